# baseline (speedup 1.0000x reference)
import functools

import jax
import jax.numpy as jnp
from jax import lax
from jax.experimental import pallas as pl
from jax.experimental.pallas import tpu as pltpu
from jax.experimental.pallas import tpu_sc as plsc

VOCAB = 1000000
EMBD = 32
NW = 32
BLK = 128
HIST = 50


@functools.cache
def _make_kernel(batch):
    n_blk = batch // BLK
    blk_per_w = n_blk // NW
    pairs = HIST // 2
    mesh = plsc.VectorSubcoreMesh(core_axis_name="c", subcore_axis_name="s")

    @functools.partial(
        pl.kernel,
        mesh=mesh,
        out_type=jax.ShapeDtypeStruct((HIST, EMBD // 8, n_blk, 8, BLK),
                                      jnp.float32),
        scratch_types=[
            pltpu.VMEM((BLK * HIST,), jnp.int32),
            pltpu.VMEM((HIST, BLK), jnp.int32),
            pltpu.VMEM((BLK, EMBD), jnp.float32),
            pltpu.VMEM((BLK, EMBD), jnp.float32),
            pltpu.VMEM((EMBD, BLK), jnp.float32),
            pltpu.VMEM((EMBD, BLK), jnp.float32),
            pltpu.SemaphoreType.DMA,
            pltpu.SemaphoreType.DMA,
            pltpu.SemaphoreType.DMA,
            pltpu.SemaphoreType.DMA,
        ],
        compiler_params=pltpu.CompilerParams(use_tc_tiling_on_sc=False,
                                             needs_layout_passes=False),
    )
    def emb_kernel(idx_hbm, table_hbm, out_hbm, idx_raw, idx_t, rows_a,
                   rows_b, out_a, out_b, gsem_a, gsem_b, wsem_a, wsem_b):
        wid = lax.axis_index("s") * 2 + lax.axis_index("c")
        iota = lax.iota(jnp.int32, 16)

        def gather_start(h, rows_v, sem):
            pltpu.make_async_copy(
                table_hbm.at[idx_t.at[h]], rows_v, sem).start()

        def gather_wait(h, rows_v, sem):
            pltpu.make_async_copy(
                table_hbm.at[idx_t.at[h]], rows_v, sem).wait()

        row_ids = [iota + (16 * q) for q in range(BLK // 16)]

        def transpose_relu(rows_v, out_v):
            for c in range(EMBD):
                col_ids = jnp.full((16,), c, jnp.int32)
                for q in range(BLK // 16):
                    vals = plsc.load_gather(rows_v, [row_ids[q], col_ids])
                    out_v[c, pl.ds(16 * q, 16)] = jnp.maximum(vals, 0.0)

        def write_start(h, jblk, out_v, sem):
            for g in range(EMBD // 8):
                pltpu.make_async_copy(
                    out_v.at[pl.ds(8 * g, 8), :],
                    out_hbm.at[h, g, jblk], sem).start()

        def write_wait(h, jblk, out_v, sem):
            for g in range(EMBD // 8):
                pltpu.make_async_copy(
                    out_v.at[pl.ds(8 * g, 8), :],
                    out_hbm.at[h, g, jblk], sem).wait()

        def jj_body(jj, jcarry):
            jblk = wid * blk_per_w + jj

            pltpu.sync_copy(idx_hbm.at[pl.ds(jblk * BLK * HIST, BLK * HIST)],
                            idx_raw)
            def idxt_body(h, carry):
                for q in range(BLK // 16):
                    base_ids = iota * HIST + (16 * HIST * q)
                    ids = plsc.load_gather(idx_raw, [base_ids + h])
                    idx_t[h, pl.ds(16 * q, 16)] = ids
                return carry

            lax.fori_loop(0, HIST, idxt_body, 0)

            gather_start(0, rows_a, gsem_a)

            def pair_body(p, carry):
                h_e = 2 * p
                h_o = h_e + 1

                @pl.when(p > 0)
                def _():
                    write_wait(h_o - 2, jblk, out_b, wsem_b)

                gather_start(h_o, rows_b, gsem_b)

                gather_wait(h_e, rows_a, gsem_a)
                transpose_relu(rows_a, out_a)
                write_start(h_e, jblk, out_a, wsem_a)

                gather_wait(h_o, rows_b, gsem_b)

                @pl.when(p < pairs - 1)
                def _():
                    write_wait(h_e, jblk, out_a, wsem_a)
                    gather_start(h_e + 2, rows_a, gsem_a)

                transpose_relu(rows_b, out_b)
                write_start(h_o, jblk, out_b, wsem_b)
                return carry

            lax.fori_loop(0, pairs, pair_body, 0)

            write_wait(HIST - 2, jblk, out_a, wsem_a)
            write_wait(HIST - 1, jblk, out_b, wsem_b)
            return jcarry

        lax.fori_loop(0, blk_per_w, jj_body, 0)

    return emb_kernel


def kernel(x, table):
    batch, hist = x.shape
    flat = x.reshape(batch * hist)
    out5 = _make_kernel(batch)(flat, table)
    return jnp.transpose(out5, (2, 4, 0, 1, 3)).reshape(batch, hist, EMBD)


# speedup vs baseline: 1.4516x; 1.4516x over previous
import functools

import jax
import jax.numpy as jnp
from jax import lax
from jax.experimental import pallas as pl
from jax.experimental.pallas import tpu as pltpu
from jax.experimental.pallas import tpu_sc as plsc

VOCAB = 1000000
EMBD = 32
NW = 32
BLK = 128
HIST = 50


@functools.cache
def _make_kernel(batch):
    n_blk = batch // BLK
    blk_per_w = n_blk // NW
    pairs = HIST // 2
    mesh = plsc.VectorSubcoreMesh(core_axis_name="c", subcore_axis_name="s")

    @functools.partial(
        pl.kernel,
        mesh=mesh,
        out_type=jax.ShapeDtypeStruct((HIST, EMBD // 8, n_blk, 8, BLK),
                                      jnp.float32),
        scratch_types=[
            pltpu.VMEM((BLK * HIST,), jnp.int32),
            pltpu.VMEM((HIST, BLK), jnp.int32),
            pltpu.VMEM((BLK, EMBD), jnp.float32),
            pltpu.VMEM((BLK, EMBD), jnp.float32),
            pltpu.VMEM((EMBD, BLK), jnp.float32),
            pltpu.VMEM((EMBD, BLK), jnp.float32),
            pltpu.SemaphoreType.DMA,
            pltpu.SemaphoreType.DMA,
            pltpu.SemaphoreType.DMA,
            pltpu.SemaphoreType.DMA,
        ],
        compiler_params=pltpu.CompilerParams(use_tc_tiling_on_sc=False,
                                             needs_layout_passes=False),
    )
    def emb_kernel(idx_hbm, table_hbm, out_hbm, idx_raw, idx_t, rows_a,
                   rows_b, out_a, out_b, gsem_a, gsem_b, wsem_a, wsem_b):
        wid = lax.axis_index("s") * 2 + lax.axis_index("c")
        iota = lax.iota(jnp.int32, 16)

        def gather_start(h, rows_v, sem):
            pltpu.make_async_copy(
                table_hbm.at[idx_t.at[h]], rows_v, sem).start()

        def gather_wait(h, rows_v, sem):
            pltpu.make_async_copy(
                table_hbm.at[idx_t.at[h]], rows_v, sem).wait()

        col_ids = [jnp.full((16,), c, jnp.int32) for c in range(EMBD)]

        def transpose_relu(rows_v, out_v):
            @plsc.parallel_loop(0, BLK, step=16, unroll=2)
            def _(q):
                vals = [
                    jnp.maximum(
                        plsc.load_gather(rows_v, [iota + q, col_ids[c]]),
                        0.0)
                    for c in range(EMBD)
                ]
                for c in range(EMBD):
                    out_v[c, pl.ds(q, 16)] = vals[c]

        def write_start(h, jblk, out_v, sem):
            for g in range(EMBD // 8):
                pltpu.make_async_copy(
                    out_v.at[pl.ds(8 * g, 8), :],
                    out_hbm.at[h, g, jblk], sem).start()

        def write_wait(h, jblk, out_v, sem):
            for g in range(EMBD // 8):
                pltpu.make_async_copy(
                    out_v.at[pl.ds(8 * g, 8), :],
                    out_hbm.at[h, g, jblk], sem).wait()

        def jj_body(jj, jcarry):
            jblk = wid * blk_per_w + jj

            pltpu.sync_copy(idx_hbm.at[pl.ds(jblk * BLK * HIST, BLK * HIST)],
                            idx_raw)
            def idxt_body(h, carry):
                for q in range(BLK // 16):
                    base_ids = iota * HIST + (16 * HIST * q)
                    ids = plsc.load_gather(idx_raw, [base_ids + h])
                    idx_t[h, pl.ds(16 * q, 16)] = ids
                return carry

            lax.fori_loop(0, HIST, idxt_body, 0)

            gather_start(0, rows_a, gsem_a)

            def pair_body(p, carry):
                h_e = 2 * p
                h_o = h_e + 1

                @pl.when(p > 0)
                def _():
                    write_wait(h_o - 2, jblk, out_b, wsem_b)

                gather_start(h_o, rows_b, gsem_b)

                gather_wait(h_e, rows_a, gsem_a)
                transpose_relu(rows_a, out_a)
                write_start(h_e, jblk, out_a, wsem_a)

                gather_wait(h_o, rows_b, gsem_b)

                @pl.when(p < pairs - 1)
                def _():
                    write_wait(h_e, jblk, out_a, wsem_a)
                    gather_start(h_e + 2, rows_a, gsem_a)

                transpose_relu(rows_b, out_b)
                write_start(h_o, jblk, out_b, wsem_b)
                return carry

            lax.fori_loop(0, pairs, pair_body, 0)

            write_wait(HIST - 2, jblk, out_a, wsem_a)
            write_wait(HIST - 1, jblk, out_b, wsem_b)
            return jcarry

        lax.fori_loop(0, blk_per_w, jj_body, 0)

    return emb_kernel


def kernel(x, table):
    batch, hist = x.shape
    flat = x.reshape(batch * hist)
    out5 = _make_kernel(batch)(flat, table)
    return jnp.transpose(out5, (2, 4, 0, 1, 3)).reshape(batch, hist, EMBD)
